# baseline (speedup 1.0000x reference)
import functools

import jax
import jax.numpy as jnp
from jax import lax
from jax.experimental import pallas as pl
from jax.experimental.pallas import tpu as pltpu
from jax.experimental.pallas import tpu_sc as plsc

N = 10000
E = 320000
DIN = 128
H = 32

NC = 2
NS = 16
NW = NC * NS

NB = 1000
EB = 1000

W_G = 512
NWIN_G = E // W_G
KMAX_G = -(-NWIN_G // NW)

NSH = 8
CG = H // 4
ESH = E // NSH
W_S = 1600

_SC_MESH = dict(core_axis_name="c", subcore_axis_name="s")


def _prep0_body(x_ref, wab_ref, a_ref, b_ref):
    ab = jnp.dot(x_ref[...], wab_ref[...], preferred_element_type=jnp.float32,
                          precision=lax.Precision.HIGHEST)
    a_ref[...] = ab[:, :H]
    b_ref[...] = ab[:, H:]


def _prep0(x, wab):
    return pl.pallas_call(
        _prep0_body,
        grid=(N // NB,),
        in_specs=[pl.BlockSpec((NB, DIN), lambda i: (i, 0)),
                  pl.BlockSpec((DIN, 2 * H), lambda i: (0, 0))],
        out_specs=[pl.BlockSpec((NB, H), lambda i: (i, 0)),
                   pl.BlockSpec((NB, H), lambda i: (i, 0))],
        out_shape=[jax.ShapeDtypeStruct((N, H), jnp.float32),
                   jax.ShapeDtypeStruct((N, H), jnp.float32)],
    )(x, wab)


def _fold_partials(p):
    m = jnp.max(p, axis=0)
    m = jnp.where(m == -jnp.inf, 0.0, m)
    return jnp.where(m > 0, m, jnp.exp(m) - 1.0)


def _prep_body(p_ref, wab_ref, a_ref, b_ref):
    m = _fold_partials(p_ref[...])
    ab = lax.dot_general(m, wab_ref[...], (((0,), (0,)), ((), ())),
                         preferred_element_type=jnp.float32,
                          precision=lax.Precision.HIGHEST)
    a_ref[...] = ab[:, :H]
    b_ref[...] = ab[:, H:]


def _prep(p, wab):
    return pl.pallas_call(
        _prep_body,
        out_shape=[jax.ShapeDtypeStruct((N, H), jnp.float32),
                   jax.ShapeDtypeStruct((N, H), jnp.float32)],
    )(p, wab)


def _edge_body(m_ref, w_ref, b_ref, f_ref):
    f_ref[...] = (jnp.dot(m_ref[...], w_ref[...],
                          preferred_element_type=jnp.float32,
                          precision=lax.Precision.HIGHEST) + b_ref[...])


def _edge_mlp(m4, w2x, b2x):
    return pl.pallas_call(
        _edge_body,
        grid=(E // 4 // EB,),
        in_specs=[pl.BlockSpec((EB, 128), lambda i: (i, 0)),
                  pl.BlockSpec((128, 128), lambda i: (0, 0)),
                  pl.BlockSpec((1, 128), lambda i: (0, 0))],
        out_specs=pl.BlockSpec((EB, 128), lambda i: (i, 0)),
        out_shape=jax.ShapeDtypeStruct((E // 4, 128), jnp.float32),
    )(m4, w2x, b2x)


def _final_body(p_ref, eye_ref, o_ref):
    m = _fold_partials(p_ref[...])
    o_ref[...] = lax.dot_general(m, eye_ref[...], (((0,), (0,)), ((), ())),
                                 preferred_element_type=jnp.float32,
                          precision=lax.Precision.HIGHEST)


def _final(p):
    eye = jnp.eye(H, dtype=jnp.float32)
    return pl.pallas_call(
        _final_body,
        out_shape=jax.ShapeDtypeStruct((N, H), jnp.float32),
    )(p, eye)


_N_STAGE = 624
_N_TAIL = N - NS * _N_STAGE


@functools.partial(
    pl.kernel,
    mesh=plsc.VectorSubcoreMesh(**_SC_MESH),
    compiler_params=pltpu.CompilerParams(use_tc_tiling_on_sc=False),
    out_type=jax.ShapeDtypeStruct((E // 4, 128), jnp.float32),
    scratch_types=[pltpu.VMEM_SHARED((N, H), jnp.float32),
                   pltpu.VMEM_SHARED((N, H), jnp.float32),
                   pltpu.VMEM((2, W_G), jnp.int32),
                   pltpu.VMEM((2, W_G), jnp.int32),
                   pltpu.VMEM((2, W_G, H), jnp.float32),
                   pltpu.VMEM((2, W_G, H), jnp.float32),
                   pltpu.VMEM((W_G // 4, 128), jnp.float32),
                   pltpu.VMEM((H,), jnp.float32),
                   pltpu.SemaphoreType.DMA,
                   pltpu.SemaphoreType.DMA,
                   pltpu.SemaphoreType.DMA,
                   pltpu.SemaphoreType.DMA],
)
def _gather_kernel(a_hbm, b_hbm, dst_hbm, src_hbm, c_hbm, m_hbm,
                   a_sp, b_sp, dst_v, src_v, arows, brows, mbuf, cbuf,
                   sem_i0, sem_i1, sem_r0, sem_r1):
    cid = lax.axis_index("c")
    sid = lax.axis_index("s")
    wid = sid * NC + cid

    r0 = pl.multiple_of(sid * _N_STAGE, 8)
    pltpu.sync_copy(a_hbm.at[pl.ds(r0, _N_STAGE)], a_sp.at[pl.ds(r0, _N_STAGE)])
    pltpu.sync_copy(b_hbm.at[pl.ds(r0, _N_STAGE)], b_sp.at[pl.ds(r0, _N_STAGE)])

    @pl.when(sid == 0)
    def _stage_tail():
        t0 = NS * _N_STAGE
        pltpu.sync_copy(a_hbm.at[pl.ds(t0, _N_TAIL)],
                        a_sp.at[pl.ds(t0, _N_TAIL)])
        pltpu.sync_copy(b_hbm.at[pl.ds(t0, _N_TAIL)],
                        b_sp.at[pl.ds(t0, _N_TAIL)])

    pltpu.sync_copy(c_hbm, cbuf)
    plsc.subcore_barrier()

    c_lo = cbuf[pl.ds(0, 16)]
    c_hi = cbuf[pl.ds(16, 16)]

    kmax = jnp.where(wid + (KMAX_G - 1) * NW < NWIN_G, KMAX_G, KMAX_G - 1)
    sem_i = [sem_i0, sem_i1]
    sem_r = [sem_r0, sem_r1]

    def _e0(k):
        return pl.multiple_of((wid + k * NW) * W_G, 32)

    def _issue_idx(k, b):
        pltpu.async_copy(dst_hbm.at[pl.ds(_e0(k), W_G)], dst_v.at[b],
                         sem_i[b])
        pltpu.async_copy(src_hbm.at[pl.ds(_e0(k), W_G)], src_v.at[b],
                         sem_i[b])

    def _wait_idx(b):
        pltpu.make_async_copy(dst_hbm.at[pl.ds(0, W_G)], dst_v.at[b],
                              sem_i[b]).wait()
        pltpu.make_async_copy(src_hbm.at[pl.ds(0, W_G)], src_v.at[b],
                              sem_i[b]).wait()

    def _issue_rows(b):
        pltpu.async_copy(a_sp.at[dst_v.at[b]], arows.at[b], sem_r[b])
        pltpu.async_copy(b_sp.at[src_v.at[b]], brows.at[b], sem_r[b])

    def _wait_rows(b):
        pltpu.make_async_copy(a_hbm.at[pl.ds(0, W_G)], arows.at[b],
                              sem_r[b]).wait()
        pltpu.make_async_copy(b_hbm.at[pl.ds(0, W_G)], brows.at[b],
                              sem_r[b]).wait()

    def _parity(bv, fn):
        @pl.when(bv == 0)
        def _b0():
            fn(0)

        @pl.when(bv == 1)
        def _b1():
            fn(1)

    _issue_idx(0, 0)
    _wait_idx(0)
    _issue_rows(0)
    _issue_idx(1, 1)

    def win(k, carry):
        b = k % 2
        nb = (k + 1) % 2

        @pl.when(k + 1 < kmax)
        def _prefetch():
            _parity(nb, lambda s: (_wait_idx(s), _issue_rows(s)))

        _parity(b, _wait_rows)

        @pl.when(k + 2 < kmax)
        def _next_idx():
            _parity(b, lambda s: _issue_idx(k + 2, s))

        def row(r, carry2):
            for cc in range(8):
                e = 4 * r + cc // 2
                col = (cc % 2) * 16
                cvec = c_lo if cc % 2 == 0 else c_hi
                v = (arows[b, e, pl.ds(col, 16)]
                     + brows[b, e, pl.ds(col, 16)] + cvec)
                mbuf[r, pl.ds(cc * 16, 16)] = jnp.maximum(v, 0.0)
            return carry2

        lax.fori_loop(0, W_G // 4, row, 0)
        pltpu.sync_copy(mbuf,
                        m_hbm.at[pl.ds(pl.multiple_of(_e0(k) // 4, 8),
                                       W_G // 4)])
        return carry

    lax.fori_loop(0, kmax, win, 0)


@functools.partial(
    pl.kernel,
    mesh=plsc.VectorSubcoreMesh(**_SC_MESH),
    compiler_params=pltpu.CompilerParams(use_tc_tiling_on_sc=False,
                                         needs_layout_passes=False),
    out_type=jax.ShapeDtypeStruct((NSH, H, N), jnp.float32),
    scratch_types=[pltpu.VMEM_SHARED((NC * 2, 2, W_S // 4, 128),
                                     jnp.float32),
                   pltpu.VMEM((CG, N), jnp.float32),
                   pltpu.VMEM((W_S,), jnp.int32),
                   pltpu.VMEM((W_S // 4, H), jnp.float32),
                   pltpu.VMEM((N,), jnp.int32),
                   pltpu.SemaphoreType.DMA],
)
def _scatter_kernel(f_hbm, dst_hbm, out_hbm, f_sp, acc, dst_v, f_v, dtmp,
                    sem_s):
    cid = lax.axis_index("c")
    sid = lax.axis_index("s")
    wid = sid * NC + cid
    es = wid % NSH
    g = wid // NSH
    slot = sid % 4

    neg = jnp.full((16,), -jnp.inf, jnp.float32)
    for c in range(CG):
        def ini(i, carry, _c=c):
            acc[_c, pl.ds(i * 16, 16)] = neg
            return carry
        lax.fori_loop(0, N // 16, ini, 0)

    cvecs = [jnp.full((16,), c, jnp.int32) for c in range(CG)]
    lanes = lax.iota(jnp.int32, 16)
    rowpat = lanes // 4
    colpat = lanes % 4

    NWIN_S = ESH // W_S

    def _stage(w, b):
        e0 = pl.multiple_of(es * ESH + w * W_S, 32)
        pltpu.async_copy(f_hbm.at[pl.ds(pl.multiple_of(e0 // 4, 8),
                                        W_S // 4)],
                         f_sp.at[slot, b], sem_s)

    @pl.when(g == 0)
    def _prologue():
        _stage(0, 0)

    def win(w, carry):
        b = w % 2
        e0 = pl.multiple_of(es * ESH + w * W_S, 32)
        pltpu.sync_copy(dst_hbm.at[pl.ds(e0, W_S)], dst_v)

        @pl.when(g == 0)
        def _wait_stage():
            pltpu.make_async_copy(f_hbm.at[pl.ds(0, W_S // 4)],
                                  f_sp.at[slot, b], sem_s).wait()

        plsc.subcore_barrier()

        @pl.when(jnp.logical_and(g == 0, w + 1 < NWIN_S))
        def _next_stage():
            _stage(w + 1, (w + 1) % 2)

        pltpu.sync_copy(f_sp.at[slot, b, :, pl.ds(g * H, H)], f_v)

        def chunk(j, carry2):
            dv = dst_v[pl.ds(j * 16, 16)]
            rowv = rowpat + j * 4
            fv = [plsc.load_gather(f_v, [rowv, colpat + 4 * c])
                  for c in range(CG)]

            plsc.store_scatter(dtmp, [dv], lanes)
            rb = plsc.load_gather(dtmp, [dv])
            nodup = jnp.all(rb == lanes)

            @pl.when(nodup)
            def _fast():
                for c in range(CG):
                    old = plsc.load_gather(acc, [cvecs[c], dv])
                    plsc.store_scatter(acc, [cvecs[c], dv],
                                       jnp.maximum(old, fv[c]))

            @pl.when(jnp.logical_not(nodup))
            def _slow():
                def cond(act):
                    return jnp.any(act)

                def body(act):
                    for c in range(CG):
                        old = plsc.load_gather(acc, [cvecs[c], dv])
                        plsc.store_scatter(acc, [cvecs[c], dv],
                                           jnp.maximum(old, fv[c]), mask=act)
                    ok = jnp.ones((16,), jnp.bool_)
                    for c in range(CG):
                        cur = plsc.load_gather(acc, [cvecs[c], dv])
                        ok = jnp.logical_and(ok, cur >= fv[c])
                    return jnp.logical_and(act, jnp.logical_not(ok))

                lax.while_loop(cond, body, jnp.ones((16,), jnp.bool_))

            return carry2

        lax.fori_loop(0, W_S // 16, chunk, 0)
        return carry

    lax.fori_loop(0, ESH // W_S, win, 0)
    pltpu.sync_copy(acc, out_hbm.at[es, pl.ds(pl.multiple_of(g * CG, 8), CG)])


def _layer_weights(W1, b1, g, be, rm, rv, W2, b2, din):
    s = g * lax.rsqrt(rv + 1e-5)
    Wt = W1 * s[None, :]
    Wa = Wt[:din] - Wt[din:]
    Wb = Wt[din:]
    wab = jnp.concatenate([Wa, Wb], axis=1)
    c = (b1 - rm) * s + be
    w2x = jnp.kron(jnp.eye(4, dtype=W2.dtype), W2)
    perm = (jnp.arange(128) % 4) * H + jnp.arange(128) // 4
    w2x = w2x[:, perm]
    b2x = jnp.repeat(b2, 4).reshape(1, 128)
    return wab, c, w2x, b2x


def kernel(x, edge_index, W1_0, b1_0, g_0, be_0, rm_0, rv_0, W2_0, b2_0,
           W1_1, b1_1, g_1, be_1, rm_1, rv_1, W2_1, b2_1,
           W1_2, b1_2, g_2, be_2, rm_2, rv_2, W2_2, b2_2):
    src = edge_index[0]
    dst = edge_index[1]

    layers = [
        (W1_0, b1_0, g_0, be_0, rm_0, rv_0, W2_0, b2_0, DIN),
        (W1_1, b1_1, g_1, be_1, rm_1, rv_1, W2_1, b2_1, H),
        (W1_2, b1_2, g_2, be_2, rm_2, rv_2, W2_2, b2_2, H),
    ]

    p = None
    for i, lw in enumerate(layers):
        wab, c, w2x, b2x = _layer_weights(*lw)
        if i == 0:
            a, b = _prep0(x, wab)
        else:
            a, b = _prep(p, wab)
        m4 = _gather_kernel(a, b, dst, src, c)
        f4 = _edge_mlp(m4, w2x, b2x)
        p = _scatter_kernel(f4, dst)

    return _final(p)


# speedup vs baseline: 1.2620x; 1.2620x over previous
import functools

import jax
import jax.numpy as jnp
from jax import lax
from jax.experimental import pallas as pl
from jax.experimental.pallas import tpu as pltpu
from jax.experimental.pallas import tpu_sc as plsc

N = 10000
E = 320000
DIN = 128
H = 32

NC = 2
NS = 16
NW = NC * NS

NB = 1000
EB = 1000

W_G = 640
NWIN_G = E // W_G
KMAX_G = -(-NWIN_G // NW)

NSH = 8
CG = H // 4
ESH = E // NSH
W_S = 1600

_SC_MESH = dict(core_axis_name="c", subcore_axis_name="s")


def _prep0_body(x_ref, wab_ref, a_ref, b_ref):
    ab = jnp.dot(x_ref[...], wab_ref[...], preferred_element_type=jnp.float32,
                          precision=lax.Precision.HIGHEST)
    a_ref[...] = ab[:, :H]
    b_ref[...] = ab[:, H:]


def _prep0(x, wab):
    return pl.pallas_call(
        _prep0_body,
        grid=(N // NB,),
        in_specs=[pl.BlockSpec((NB, DIN), lambda i: (i, 0)),
                  pl.BlockSpec((DIN, 2 * H), lambda i: (0, 0))],
        out_specs=[pl.BlockSpec((NB, H), lambda i: (i, 0)),
                   pl.BlockSpec((NB, H), lambda i: (i, 0))],
        out_shape=[jax.ShapeDtypeStruct((N, H), jnp.float32),
                   jax.ShapeDtypeStruct((N, H), jnp.float32)],
    )(x, wab)


def _fold_partials(p):
    m = jnp.max(p, axis=0)
    m = jnp.where(m == -jnp.inf, 0.0, m)
    return jnp.where(m > 0, m, jnp.exp(m) - 1.0)


def _prep_body(p_ref, wab_ref, a_ref, b_ref):
    m = _fold_partials(p_ref[...])
    ab = lax.dot_general(m, wab_ref[...], (((0,), (0,)), ((), ())),
                         preferred_element_type=jnp.float32,
                          precision=lax.Precision.HIGHEST)
    a_ref[...] = ab[:, :H]
    b_ref[...] = ab[:, H:]


def _prep(p, wab):
    return pl.pallas_call(
        _prep_body,
        out_shape=[jax.ShapeDtypeStruct((N, H), jnp.float32),
                   jax.ShapeDtypeStruct((N, H), jnp.float32)],
    )(p, wab)


def _edge_body(m_ref, w_ref, b_ref, f_ref):
    f_ref[...] = (jnp.dot(m_ref[...], w_ref[...],
                          preferred_element_type=jnp.float32,
                          precision=lax.Precision.HIGHEST) + b_ref[...])


def _edge_mlp(m4, w2x, b2x):
    return pl.pallas_call(
        _edge_body,
        grid=(E // 4 // EB,),
        in_specs=[pl.BlockSpec((EB, 128), lambda i: (i, 0)),
                  pl.BlockSpec((128, 128), lambda i: (0, 0)),
                  pl.BlockSpec((1, 128), lambda i: (0, 0))],
        out_specs=pl.BlockSpec((EB, 128), lambda i: (i, 0)),
        out_shape=jax.ShapeDtypeStruct((E // 4, 128), jnp.float32),
    )(m4, w2x, b2x)


def _final_body(p_ref, eye_ref, o_ref):
    m = _fold_partials(p_ref[...])
    o_ref[...] = lax.dot_general(m, eye_ref[...], (((0,), (0,)), ((), ())),
                                 preferred_element_type=jnp.float32,
                          precision=lax.Precision.HIGHEST)


def _final(p):
    eye = jnp.eye(H, dtype=jnp.float32)
    return pl.pallas_call(
        _final_body,
        out_shape=jax.ShapeDtypeStruct((N, H), jnp.float32),
    )(p, eye)


_N_STAGE = 624
_N_TAIL = N - NS * _N_STAGE


@functools.partial(
    pl.kernel,
    mesh=plsc.VectorSubcoreMesh(**_SC_MESH),
    compiler_params=pltpu.CompilerParams(use_tc_tiling_on_sc=False),
    out_type=jax.ShapeDtypeStruct((E // 4, 128), jnp.float32),
    scratch_types=[pltpu.VMEM_SHARED((N, H), jnp.float32),
                   pltpu.VMEM_SHARED((N, H), jnp.float32),
                   pltpu.VMEM((W_G,), jnp.int32),
                   pltpu.VMEM((W_G,), jnp.int32),
                   pltpu.VMEM((W_G, H), jnp.float32),
                   pltpu.VMEM((W_G, H), jnp.float32),
                   pltpu.VMEM((W_G // 4, 128), jnp.float32),
                   pltpu.VMEM((H,), jnp.float32),
                   pltpu.SemaphoreType.DMA,
                   pltpu.SemaphoreType.DMA],
)
def _gather_kernel(a_hbm, b_hbm, dst_hbm, src_hbm, c_hbm, m_hbm,
                   a_sp, b_sp, dst_v, src_v, arows, brows, mbuf, cbuf,
                   sem_a, sem_b):
    cid = lax.axis_index("c")
    sid = lax.axis_index("s")
    wid = sid * NC + cid

    r0 = pl.multiple_of(sid * _N_STAGE, 8)
    pltpu.sync_copy(a_hbm.at[pl.ds(r0, _N_STAGE)], a_sp.at[pl.ds(r0, _N_STAGE)])
    pltpu.sync_copy(b_hbm.at[pl.ds(r0, _N_STAGE)], b_sp.at[pl.ds(r0, _N_STAGE)])

    @pl.when(sid == 0)
    def _stage_tail():
        t0 = NS * _N_STAGE
        pltpu.sync_copy(a_hbm.at[pl.ds(t0, _N_TAIL)],
                        a_sp.at[pl.ds(t0, _N_TAIL)])
        pltpu.sync_copy(b_hbm.at[pl.ds(t0, _N_TAIL)],
                        b_sp.at[pl.ds(t0, _N_TAIL)])

    pltpu.sync_copy(c_hbm, cbuf)
    plsc.subcore_barrier()

    c_lo = cbuf[pl.ds(0, 16)]
    c_hi = cbuf[pl.ds(16, 16)]

    def win(k, carry):
        t = wid + k * NW

        @pl.when(t < NWIN_G)
        def _do():
            e0 = pl.multiple_of(t * W_G, 32)
            pltpu.sync_copy(dst_hbm.at[pl.ds(e0, W_G)], dst_v)
            pltpu.sync_copy(src_hbm.at[pl.ds(e0, W_G)], src_v)
            ca = pltpu.async_copy(a_sp.at[dst_v], arows, sem_a)
            cb = pltpu.async_copy(b_sp.at[src_v], brows, sem_b)
            ca.wait()
            cb.wait()

            def row(r, carry2):
                for cc in range(8):
                    e = 4 * r + cc // 2
                    col = (cc % 2) * 16
                    cvec = c_lo if cc % 2 == 0 else c_hi
                    v = (arows[e, pl.ds(col, 16)] + brows[e, pl.ds(col, 16)]
                         + cvec)
                    mbuf[r, pl.ds(cc * 16, 16)] = jnp.maximum(v, 0.0)
                return carry2

            lax.fori_loop(0, W_G // 4, row, 0)
            pltpu.sync_copy(mbuf,
                            m_hbm.at[pl.ds(pl.multiple_of(e0 // 4, 8),
                                           W_G // 4)])

        return carry

    lax.fori_loop(0, KMAX_G, win, 0)


@functools.partial(
    pl.kernel,
    mesh=plsc.VectorSubcoreMesh(**_SC_MESH),
    compiler_params=pltpu.CompilerParams(use_tc_tiling_on_sc=False,
                                         needs_layout_passes=False),
    out_type=jax.ShapeDtypeStruct((NSH, H, N), jnp.float32),
    scratch_types=[pltpu.VMEM_SHARED((NC * 2, W_S // 4, 128), jnp.float32),
                   pltpu.VMEM((CG, N), jnp.float32),
                   pltpu.VMEM((W_S,), jnp.int32),
                   pltpu.VMEM((W_S // 4, H), jnp.float32),
                   pltpu.VMEM((N,), jnp.int32)],
)
def _scatter_kernel(f_hbm, dst_hbm, out_hbm, f_sp, acc, dst_v, f_v, dtmp):
    cid = lax.axis_index("c")
    sid = lax.axis_index("s")
    wid = sid * NC + cid
    es = wid % NSH
    g = wid // NSH
    slot = sid % 4

    neg = jnp.full((16,), -jnp.inf, jnp.float32)
    for c in range(CG):
        def ini(i, carry, _c=c):
            acc[_c, pl.ds(i * 16, 16)] = neg
            return carry
        lax.fori_loop(0, N // 16, ini, 0)

    cvecs = [jnp.full((16,), c, jnp.int32) for c in range(CG)]
    lanes = lax.iota(jnp.int32, 16)
    rowpat = lanes // 4
    colpat = lanes % 4

    def win(w, carry):
        e0 = pl.multiple_of(es * ESH + w * W_S, 32)
        pltpu.sync_copy(dst_hbm.at[pl.ds(e0, W_S)], dst_v)

        @pl.when(g == 0)
        def _stage():
            pltpu.sync_copy(f_hbm.at[pl.ds(pl.multiple_of(e0 // 4, 8),
                                           W_S // 4)],
                            f_sp.at[slot])

        plsc.subcore_barrier()
        pltpu.sync_copy(f_sp.at[slot, :, pl.ds(g * H, H)], f_v)

        def _slow(dv, fv):
            def cond(act):
                return jnp.any(act)

            def body(act):
                for c in range(CG):
                    old = plsc.load_gather(acc, [cvecs[c], dv])
                    plsc.store_scatter(acc, [cvecs[c], dv],
                                       jnp.maximum(old, fv[c]), mask=act)
                ok = jnp.ones((16,), jnp.bool_)
                for c in range(CG):
                    cur = plsc.load_gather(acc, [cvecs[c], dv])
                    ok = jnp.logical_and(ok, cur >= fv[c])
                return jnp.logical_and(act, jnp.logical_not(ok))

            lax.while_loop(cond, body, jnp.ones((16,), jnp.bool_))

        lanes16 = lanes + 16

        def chunkpair(j2, carry2):
            j0 = 2 * j2
            dv0 = dst_v[pl.ds(pl.multiple_of(j0 * 16, 16), 16)]
            dv1 = dst_v[pl.ds(pl.multiple_of(j0 * 16 + 16, 16), 16)]
            rowv0 = rowpat + j0 * 4
            rowv1 = rowpat + j0 * 4 + 4
            fv0 = [plsc.load_gather(f_v, [rowv0, colpat + 4 * c])
                   for c in range(CG)]
            fv1 = [plsc.load_gather(f_v, [rowv1, colpat + 4 * c])
                   for c in range(CG)]

            plsc.store_scatter(dtmp, [dv0], lanes)
            plsc.store_scatter(dtmp, [dv1], lanes16)
            rb0 = plsc.load_gather(dtmp, [dv0])
            rb1 = plsc.load_gather(dtmp, [dv1])
            nodup = jnp.all(jnp.logical_and(rb0 == lanes, rb1 == lanes16))

            @pl.when(nodup)
            def _fast():
                for c in range(CG):
                    old0 = plsc.load_gather(acc, [cvecs[c], dv0])
                    old1 = plsc.load_gather(acc, [cvecs[c], dv1])
                    plsc.store_scatter(acc, [cvecs[c], dv0],
                                       jnp.maximum(old0, fv0[c]))
                    plsc.store_scatter(acc, [cvecs[c], dv1],
                                       jnp.maximum(old1, fv1[c]))

            @pl.when(jnp.logical_not(nodup))
            def _dup():
                _slow(dv0, fv0)
                _slow(dv1, fv1)

            return carry2

        lax.fori_loop(0, W_S // 32, chunkpair, 0)
        plsc.subcore_barrier()
        return carry

    lax.fori_loop(0, ESH // W_S, win, 0)
    pltpu.sync_copy(acc, out_hbm.at[es, pl.ds(pl.multiple_of(g * CG, 8), CG)])


def _layer_weights(W1, b1, g, be, rm, rv, W2, b2, din):
    s = g * lax.rsqrt(rv + 1e-5)
    Wt = W1 * s[None, :]
    Wa = Wt[:din] - Wt[din:]
    Wb = Wt[din:]
    wab = jnp.concatenate([Wa, Wb], axis=1)
    c = (b1 - rm) * s + be
    w2x = jnp.kron(jnp.eye(4, dtype=W2.dtype), W2)
    perm = (jnp.arange(128) % 4) * H + jnp.arange(128) // 4
    w2x = w2x[:, perm]
    b2x = jnp.repeat(b2, 4).reshape(1, 128)
    return wab, c, w2x, b2x


def kernel(x, edge_index, W1_0, b1_0, g_0, be_0, rm_0, rv_0, W2_0, b2_0,
           W1_1, b1_1, g_1, be_1, rm_1, rv_1, W2_1, b2_1,
           W1_2, b1_2, g_2, be_2, rm_2, rv_2, W2_2, b2_2):
    src = edge_index[0]
    dst = edge_index[1]

    layers = [
        (W1_0, b1_0, g_0, be_0, rm_0, rv_0, W2_0, b2_0, DIN),
        (W1_1, b1_1, g_1, be_1, rm_1, rv_1, W2_1, b2_1, H),
        (W1_2, b1_2, g_2, be_2, rm_2, rv_2, W2_2, b2_2, H),
    ]

    p = None
    for i, lw in enumerate(layers):
        wab, c, w2x, b2x = _layer_weights(*lw)
        if i == 0:
            a, b = _prep0(x, wab)
        else:
            a, b = _prep(p, wab)
        m4 = _gather_kernel(a, b, dst, src, c)
        f4 = _edge_mlp(m4, w2x, b2x)
        p = _scatter_kernel(f4, dst)

    return _final(p)


# speedup vs baseline: 1.3320x; 1.0554x over previous
import functools

import jax
import jax.numpy as jnp
from jax import lax
from jax.experimental import pallas as pl
from jax.experimental.pallas import tpu as pltpu
from jax.experimental.pallas import tpu_sc as plsc

N = 10000
E = 320000
DIN = 128
H = 32

NC = 2
NS = 16
NW = NC * NS

NB = 1000
EB = 1000

W_G = 640
NWIN_G = E // W_G
KMAX_G = -(-NWIN_G // NW)

NSH = 8
CG = H // 4
ESH = E // NSH
W_S = 1600

_SC_MESH = dict(core_axis_name="c", subcore_axis_name="s")


def _prep0_body(x_ref, wab_ref, a_ref, b_ref):
    ab = jnp.dot(x_ref[...], wab_ref[...], preferred_element_type=jnp.float32,
                          precision=lax.Precision.HIGHEST)
    a_ref[...] = ab[:, :H]
    b_ref[...] = ab[:, H:]


def _prep0(x, wab):
    return pl.pallas_call(
        _prep0_body,
        grid=(N // NB,),
        in_specs=[pl.BlockSpec((NB, DIN), lambda i: (i, 0)),
                  pl.BlockSpec((DIN, 2 * H), lambda i: (0, 0))],
        out_specs=[pl.BlockSpec((NB, H), lambda i: (i, 0)),
                   pl.BlockSpec((NB, H), lambda i: (i, 0))],
        out_shape=[jax.ShapeDtypeStruct((N, H), jnp.float32),
                   jax.ShapeDtypeStruct((N, H), jnp.float32)],
    )(x, wab)


def _fold_partials(p):
    m = jnp.max(p, axis=0)
    m = jnp.where(m == -jnp.inf, 0.0, m)
    return jnp.where(m > 0, m, jnp.exp(m) - 1.0)


def _prep_body(p_ref, wab_ref, a_ref, b_ref):
    m = _fold_partials(p_ref[...])
    ab = lax.dot_general(m, wab_ref[...], (((0,), (0,)), ((), ())),
                         preferred_element_type=jnp.float32,
                          precision=lax.Precision.HIGHEST)
    a_ref[...] = ab[:, :H]
    b_ref[...] = ab[:, H:]


def _prep(p, wab):
    return pl.pallas_call(
        _prep_body,
        out_shape=[jax.ShapeDtypeStruct((N, H), jnp.float32),
                   jax.ShapeDtypeStruct((N, H), jnp.float32)],
    )(p, wab)


def _edge_body(m_ref, w_ref, b_ref, f_ref):
    f_ref[...] = (jnp.dot(m_ref[...], w_ref[...],
                          preferred_element_type=jnp.float32,
                          precision=lax.Precision.HIGHEST) + b_ref[...])


def _edge_mlp(m4, w2x, b2x):
    return pl.pallas_call(
        _edge_body,
        grid=(E // 4 // EB,),
        in_specs=[pl.BlockSpec((EB, 128), lambda i: (i, 0)),
                  pl.BlockSpec((128, 128), lambda i: (0, 0)),
                  pl.BlockSpec((1, 128), lambda i: (0, 0))],
        out_specs=pl.BlockSpec((EB, 128), lambda i: (i, 0)),
        out_shape=jax.ShapeDtypeStruct((E // 4, 128), jnp.float32),
    )(m4, w2x, b2x)


def _final_body(p_ref, eye_ref, o_ref):
    m = _fold_partials(p_ref[...])
    o_ref[...] = lax.dot_general(m, eye_ref[...], (((0,), (0,)), ((), ())),
                                 preferred_element_type=jnp.float32,
                          precision=lax.Precision.HIGHEST)


def _final(p):
    eye = jnp.eye(H, dtype=jnp.float32)
    return pl.pallas_call(
        _final_body,
        out_shape=jax.ShapeDtypeStruct((N, H), jnp.float32),
    )(p, eye)


_N_STAGE = 624
_N_TAIL = N - NS * _N_STAGE


@functools.partial(
    pl.kernel,
    mesh=plsc.VectorSubcoreMesh(**_SC_MESH),
    compiler_params=pltpu.CompilerParams(use_tc_tiling_on_sc=False),
    out_type=jax.ShapeDtypeStruct((E // 4, 128), jnp.float32),
    scratch_types=[pltpu.VMEM_SHARED((N, H), jnp.float32),
                   pltpu.VMEM_SHARED((N, H), jnp.float32),
                   pltpu.VMEM((W_G,), jnp.int32),
                   pltpu.VMEM((W_G,), jnp.int32),
                   pltpu.VMEM((W_G, H), jnp.float32),
                   pltpu.VMEM((W_G, H), jnp.float32),
                   pltpu.VMEM((W_G // 4, 128), jnp.float32),
                   pltpu.VMEM((H,), jnp.float32),
                   pltpu.SemaphoreType.DMA,
                   pltpu.SemaphoreType.DMA],
)
def _gather_kernel(a_hbm, b_hbm, dst_hbm, src_hbm, c_hbm, m_hbm,
                   a_sp, b_sp, dst_v, src_v, arows, brows, mbuf, cbuf,
                   sem_a, sem_b):
    cid = lax.axis_index("c")
    sid = lax.axis_index("s")
    wid = sid * NC + cid

    r0 = pl.multiple_of(sid * _N_STAGE, 8)
    pltpu.sync_copy(a_hbm.at[pl.ds(r0, _N_STAGE)], a_sp.at[pl.ds(r0, _N_STAGE)])
    pltpu.sync_copy(b_hbm.at[pl.ds(r0, _N_STAGE)], b_sp.at[pl.ds(r0, _N_STAGE)])

    @pl.when(sid == 0)
    def _stage_tail():
        t0 = NS * _N_STAGE
        pltpu.sync_copy(a_hbm.at[pl.ds(t0, _N_TAIL)],
                        a_sp.at[pl.ds(t0, _N_TAIL)])
        pltpu.sync_copy(b_hbm.at[pl.ds(t0, _N_TAIL)],
                        b_sp.at[pl.ds(t0, _N_TAIL)])

    pltpu.sync_copy(c_hbm, cbuf)
    plsc.subcore_barrier()

    c_lo = cbuf[pl.ds(0, 16)]
    c_hi = cbuf[pl.ds(16, 16)]

    def win(k, carry):
        t = wid + k * NW

        @pl.when(t < NWIN_G)
        def _do():
            e0 = pl.multiple_of(t * W_G, 32)
            pltpu.sync_copy(dst_hbm.at[pl.ds(e0, W_G)], dst_v)
            pltpu.sync_copy(src_hbm.at[pl.ds(e0, W_G)], src_v)
            ca = pltpu.async_copy(a_sp.at[dst_v], arows, sem_a)
            cb = pltpu.async_copy(b_sp.at[src_v], brows, sem_b)
            ca.wait()
            cb.wait()

            def row(r2, carry2):
                for u in range(2):
                    r = 2 * r2 + u
                    for cc in range(8):
                        e = 4 * r + cc // 2
                        col = (cc % 2) * 16
                        cvec = c_lo if cc % 2 == 0 else c_hi
                        v = (arows[e, pl.ds(col, 16)]
                             + brows[e, pl.ds(col, 16)] + cvec)
                        mbuf[r, pl.ds(cc * 16, 16)] = jnp.maximum(v, 0.0)
                return carry2

            lax.fori_loop(0, W_G // 8, row, 0)
            pltpu.sync_copy(mbuf,
                            m_hbm.at[pl.ds(pl.multiple_of(e0 // 4, 8),
                                           W_G // 4)])

        return carry

    lax.fori_loop(0, KMAX_G, win, 0)


@functools.partial(
    pl.kernel,
    mesh=plsc.VectorSubcoreMesh(**_SC_MESH),
    compiler_params=pltpu.CompilerParams(use_tc_tiling_on_sc=False,
                                         needs_layout_passes=False),
    out_type=jax.ShapeDtypeStruct((NSH, H, N), jnp.float32),
    scratch_types=[pltpu.VMEM_SHARED((NC * 2, W_S // 4, 128), jnp.float32),
                   pltpu.VMEM((CG, N), jnp.float32),
                   pltpu.VMEM((W_S,), jnp.int32),
                   pltpu.VMEM((W_S // 4, H), jnp.float32),
                   pltpu.VMEM((N,), jnp.int32)],
)
def _scatter_kernel(f_hbm, dst_hbm, out_hbm, f_sp, acc, dst_v, f_v, dtmp):
    cid = lax.axis_index("c")
    sid = lax.axis_index("s")
    wid = sid * NC + cid
    es = wid % NSH
    g = wid // NSH
    slot = sid % 4

    neg = jnp.full((16,), -jnp.inf, jnp.float32)
    for c in range(CG):
        def ini(i, carry, _c=c):
            acc[_c, pl.ds(i * 16, 16)] = neg
            return carry
        lax.fori_loop(0, N // 16, ini, 0)

    cvecs = [jnp.full((16,), c, jnp.int32) for c in range(CG)]
    lanes = lax.iota(jnp.int32, 16)
    rowpat = lanes // 4
    colpat = lanes % 4

    def win(w, carry):
        e0 = pl.multiple_of(es * ESH + w * W_S, 32)
        pltpu.sync_copy(dst_hbm.at[pl.ds(e0, W_S)], dst_v)

        @pl.when(g == 0)
        def _stage():
            pltpu.sync_copy(f_hbm.at[pl.ds(pl.multiple_of(e0 // 4, 8),
                                           W_S // 4)],
                            f_sp.at[slot])

        plsc.subcore_barrier()
        pltpu.sync_copy(f_sp.at[slot, :, pl.ds(g * H, H)], f_v)

        def _slow(dv, fv):
            def cond(act):
                return jnp.any(act)

            def body(act):
                for c in range(CG):
                    old = plsc.load_gather(acc, [cvecs[c], dv])
                    plsc.store_scatter(acc, [cvecs[c], dv],
                                       jnp.maximum(old, fv[c]), mask=act)
                ok = jnp.ones((16,), jnp.bool_)
                for c in range(CG):
                    cur = plsc.load_gather(acc, [cvecs[c], dv])
                    ok = jnp.logical_and(ok, cur >= fv[c])
                return jnp.logical_and(act, jnp.logical_not(ok))

            lax.while_loop(cond, body, jnp.ones((16,), jnp.bool_))

        lane_ids = [lanes, lanes + 16, lanes + 32, lanes + 48]

        def chunkquad(j4, carry2):
            j0 = 4 * j4
            dv = [dst_v[pl.ds(pl.multiple_of((j0 + i) * 16, 16), 16)]
                  for i in range(4)]
            rowv = [rowpat + (j0 + i) * 4 for i in range(4)]
            fv = [[plsc.load_gather(f_v, [rowv[i], colpat + 4 * c])
                   for c in range(CG)] for i in range(4)]

            for i in range(4):
                plsc.store_scatter(dtmp, [dv[i]], lane_ids[i])
            rb = [plsc.load_gather(dtmp, [dv[i]]) for i in range(4)]
            okv = [rb[i] == lane_ids[i] for i in range(4)]
            nodup01 = jnp.all(jnp.logical_and(okv[0], okv[1]))
            nodup23 = jnp.all(jnp.logical_and(okv[2], okv[3]))
            nodup = jnp.logical_and(nodup01, nodup23)

            @pl.when(nodup)
            def _fast():
                for c in range(CG):
                    olds = [plsc.load_gather(acc, [cvecs[c], dv[i]])
                            for i in range(4)]
                    for i in range(4):
                        plsc.store_scatter(acc, [cvecs[c], dv[i]],
                                           jnp.maximum(olds[i], fv[i][c]))

            @pl.when(jnp.logical_not(nodup))
            def _dup():
                def pair(i0, i1):
                    nd = jnp.all(jnp.logical_and(okv[i0], okv[i1]))

                    @pl.when(nd)
                    def _pfast():
                        for c in range(CG):
                            o0 = plsc.load_gather(acc, [cvecs[c], dv[i0]])
                            o1 = plsc.load_gather(acc, [cvecs[c], dv[i1]])
                            plsc.store_scatter(acc, [cvecs[c], dv[i0]],
                                               jnp.maximum(o0, fv[i0][c]))
                            plsc.store_scatter(acc, [cvecs[c], dv[i1]],
                                               jnp.maximum(o1, fv[i1][c]))

                    @pl.when(jnp.logical_not(nd))
                    def _pslow():
                        _slow(dv[i0], fv[i0])
                        _slow(dv[i1], fv[i1])

                pair(0, 1)
                pair(2, 3)

            return carry2

        lax.fori_loop(0, W_S // 64, chunkquad, 0)
        plsc.subcore_barrier()
        return carry

    lax.fori_loop(0, ESH // W_S, win, 0)
    pltpu.sync_copy(acc, out_hbm.at[es, pl.ds(pl.multiple_of(g * CG, 8), CG)])


def _layer_weights(W1, b1, g, be, rm, rv, W2, b2, din):
    s = g * lax.rsqrt(rv + 1e-5)
    Wt = W1 * s[None, :]
    Wa = Wt[:din] - Wt[din:]
    Wb = Wt[din:]
    wab = jnp.concatenate([Wa, Wb], axis=1)
    c = (b1 - rm) * s + be
    w2x = jnp.kron(jnp.eye(4, dtype=W2.dtype), W2)
    perm = (jnp.arange(128) % 4) * H + jnp.arange(128) // 4
    w2x = w2x[:, perm]
    b2x = jnp.repeat(b2, 4).reshape(1, 128)
    return wab, c, w2x, b2x


def kernel(x, edge_index, W1_0, b1_0, g_0, be_0, rm_0, rv_0, W2_0, b2_0,
           W1_1, b1_1, g_1, be_1, rm_1, rv_1, W2_1, b2_1,
           W1_2, b1_2, g_2, be_2, rm_2, rv_2, W2_2, b2_2):
    src = edge_index[0]
    dst = edge_index[1]

    layers = [
        (W1_0, b1_0, g_0, be_0, rm_0, rv_0, W2_0, b2_0, DIN),
        (W1_1, b1_1, g_1, be_1, rm_1, rv_1, W2_1, b2_1, H),
        (W1_2, b1_2, g_2, be_2, rm_2, rv_2, W2_2, b2_2, H),
    ]

    p = None
    for i, lw in enumerate(layers):
        wab, c, w2x, b2x = _layer_weights(*lw)
        if i == 0:
            a, b = _prep0(x, wab)
        else:
            a, b = _prep(p, wab)
        m4 = _gather_kernel(a, b, dst, src, c)
        f4 = _edge_mlp(m4, w2x, b2x)
        p = _scatter_kernel(f4, dst)

    return _final(p)
